# baseline (speedup 1.0000x reference)
import functools

import jax
import jax.numpy as jnp
from jax import lax
from jax.experimental import pallas as pl
from jax.experimental.pallas import tpu as pltpu
from jax.experimental.pallas import tpu_sc as plsc

N_NODES = 10000
DEG = 32
D = 128
OUT_DIM = 128
N_CLASSES = 40

NC = 2
NS = 16
NW = NC * NS
BP = 10240
BPW = BP // NW
NB = 64
NBURST = BPW // NB
NROW = NB // 4


def _sc_sage_body(inputs_hbm, adjp_hbm, feat_hbm, self_hbm, sum_hbm,
                  iv, av, avc, sv, nbuf, didx, stage, acc, gsem, ssem, wsem):
    c = lax.axis_index("c")
    s = lax.axis_index("s")
    wid = s * NC + c
    base = wid * BPW
    lbase = s * BPW

    pltpu.sync_copy(inputs_hbm.at[pl.ds(base, BPW)], iv)

    for q in range(8):
        didx[0, pl.ds(q * 16, 16)] = jnp.full((16,), q // 2, jnp.int32)

    for q in range(8):
        stage[0, pl.ds(q * 16, 16)] = jnp.zeros((16,), jnp.float32)
    for i in range(1, NB):
        for q in range(8):
            stage[i, pl.ds(q * 16, 16)] = stage[0, pl.ds(q * 16, 16)]
    for k in range(NBURST):
        pltpu.sync_copy(stage, acc.at[pl.ds(lbase + k * NB, NB)])

    def burst(bi, carry):
        off = bi * NB
        idx = iv.at[pl.ds(off, NB)]
        c1 = pltpu.async_copy(adjp_hbm.at[idx], av, gsem)
        c2 = pltpu.async_copy(feat_hbm.at[idx], sv, gsem)
        c1.wait()
        c2.wait()
        wc = pltpu.async_copy(sv, self_hbm.at[pl.ds(base + off, NB)], wsem)

        for i in range(NB):
            r, h = divmod(i, 4)
            avc[r, pl.ds(32 * h, 16)] = av[i, pl.ds(0, 16)]
            avc[r, pl.ds(32 * h + 16, 16)] = av[i, pl.ds(16, 16)]

        pltpu.async_copy(feat_hbm.at[avc.at[0]], nbuf.at[0], gsem)

        def pair(g, carry):
            for b in range(2):
                rr = 2 * g + b
                pltpu.make_async_copy(
                    feat_hbm.at[avc.at[0]], nbuf.at[b], gsem).wait()
                dst = acc.at[pl.ds(lbase + off + rr * 4, 4)]
                pltpu.async_copy(nbuf.at[b], dst.at[didx.at[0]], ssem,
                                 add=True)

                @pl.when(rr + 1 < NROW)
                def _():
                    @pl.when(rr >= 1)
                    def _():
                        pltpu.make_async_copy(
                            nbuf.at[1 - b],
                            acc.at[pl.ds(lbase, 4)].at[didx.at[0]],
                            ssem).wait()
                    pltpu.async_copy(
                        feat_hbm.at[avc.at[rr + 1]], nbuf.at[1 - b], gsem)
            return carry

        lax.fori_loop(0, NROW // 2, pair, 0)
        for _ in range(2):
            pltpu.make_async_copy(
                nbuf.at[0], acc.at[pl.ds(lbase, 4)].at[didx.at[0]],
                ssem).wait()
        wc.wait()
        return carry

    lax.fori_loop(0, NBURST, burst, 0)

    for k in range(NBURST):
        pltpu.sync_copy(acc.at[pl.ds(lbase + k * NB, NB)], stage)
        pltpu.sync_copy(stage, sum_hbm.at[pl.ds(base + k * NB, NB)])


@functools.lru_cache(maxsize=1)
def _build_sc_kernel():
    mesh = plsc.VectorSubcoreMesh(core_axis_name="c", subcore_axis_name="s")
    return pl.kernel(
        _sc_sage_body,
        out_type=[
            jax.ShapeDtypeStruct((BP, D), jnp.float32),
            jax.ShapeDtypeStruct((BP, D), jnp.float32),
        ],
        mesh=mesh,
        scratch_types=[
            pltpu.VMEM((BPW,), jnp.int32),
            pltpu.VMEM((NB, 128), jnp.int32),
            pltpu.VMEM((NROW, 128), jnp.int32),
            pltpu.VMEM((NB, D), jnp.float32),
            pltpu.VMEM((2, 128, D), jnp.float32),
            pltpu.VMEM((1, 128), jnp.int32),
            pltpu.VMEM((NB, D), jnp.float32),
            pltpu.VMEM_SHARED((NS * BPW, D), jnp.float32),
            pltpu.SemaphoreType.DMA,
            pltpu.SemaphoreType.DMA,
            pltpu.SemaphoreType.DMA,
        ],
    )


_RB = 512


def _tc_body(self_ref, sum_ref, w1a_ref, w1s_ref, b1_ref, w3_ref, b3_ref, out_ref):
    x = jnp.dot(self_ref[...], w1a_ref[...], preferred_element_type=jnp.float32)
    x = x + jnp.dot(sum_ref[...], w1s_ref[...], preferred_element_type=jnp.float32)
    x = jnp.maximum(x + b1_ref[...], 0.0)
    l = jnp.dot(x, w3_ref[...], preferred_element_type=jnp.float32) + b3_ref[...]
    ss = jnp.sum(l * l, axis=1, keepdims=True)
    denom = jnp.maximum(jnp.sqrt(ss), 1e-12)
    out_ref[...] = l / denom


def kernel(inputs, adj, feat_data, W1, b1, W3, b3):
    B = inputs.shape[0]
    inputs_p = jnp.concatenate(
        [inputs.astype(jnp.int32), jnp.zeros((BP - B,), jnp.int32)])
    adj_p = jnp.pad(adj, ((0, 0), (0, 128 - DEG)))

    self_feat, sums = _build_sc_kernel()(inputs_p, adj_p, feat_data)

    w1a_t = W1[:, :D].T
    w1s_t = (W1[:, D:] * (1.0 / DEG)).T
    w3_t = jnp.pad(W3.T, ((0, 0), (0, 128 - N_CLASSES)))
    b1_r = b1.reshape(1, OUT_DIM)
    b3_r = jnp.pad(b3, (0, 128 - N_CLASSES)).reshape(1, 128)

    logits = pl.pallas_call(
        _tc_body,
        out_shape=jax.ShapeDtypeStruct((BP, 128), jnp.float32),
        grid=(BP // _RB,),
        in_specs=[
            pl.BlockSpec((_RB, D), lambda i: (i, 0)),
            pl.BlockSpec((_RB, D), lambda i: (i, 0)),
            pl.BlockSpec((D, OUT_DIM), lambda i: (0, 0)),
            pl.BlockSpec((D, OUT_DIM), lambda i: (0, 0)),
            pl.BlockSpec((1, OUT_DIM), lambda i: (0, 0)),
            pl.BlockSpec((OUT_DIM, 128), lambda i: (0, 0)),
            pl.BlockSpec((1, 128), lambda i: (0, 0)),
        ],
        out_specs=pl.BlockSpec((_RB, 128), lambda i: (i, 0)),
    )(self_feat, sums, w1a_t, w1s_t, b1_r, w3_t, b3_r)

    return logits[:B, :N_CLASSES]


# speedup vs baseline: 1.2357x; 1.2357x over previous
import functools

import jax
import jax.numpy as jnp
from jax import lax
from jax.experimental import pallas as pl
from jax.experimental.pallas import tpu as pltpu
from jax.experimental.pallas import tpu_sc as plsc

N_NODES = 10000
DEG = 32
D = 128
OUT_DIM = 128
N_CLASSES = 40

NC = 2
NS = 16
NW = NC * NS
BP = 10240
BPW = BP // NW
NB = 64
NBURST = BPW // NB
NSTREAM = BPW // 4
NRING = 4


def _sc_sage_body(inputs_hbm, adjp_hbm, feat_hbm, self_hbm, sum_hbm,
                  iv, av, avc, sv, nbuf, sumbuf, gsem, wsem):
    c = lax.axis_index("c")
    s = lax.axis_index("s")
    wid = s * NC + c
    base = wid * BPW

    pltpu.sync_copy(inputs_hbm.at[pl.ds(base, BPW)], iv)

    def hdr(bi, carry):
        off = bi * NB
        idx = iv.at[pl.ds(off, NB)]
        c1 = pltpu.async_copy(adjp_hbm.at[idx], av, gsem)
        c2 = pltpu.async_copy(feat_hbm.at[idx], sv, gsem)
        c1.wait()
        c2.wait()
        wc = pltpu.async_copy(sv, self_hbm.at[pl.ds(base + off, NB)], wsem)
        arow = bi * (NB // 4)
        for i in range(NB):
            r, h = divmod(i, 4)
            avc[arow + r, pl.ds(32 * h, 16)] = av[i, pl.ds(0, 16)]
            avc[arow + r, pl.ds(32 * h + 16, 16)] = av[i, pl.ds(16, 16)]
        wc.wait()
        return carry

    lax.fori_loop(0, NBURST, hdr, 0)

    for t in range(NRING - 1):
        pltpu.async_copy(feat_hbm.at[avc.at[t]], nbuf.at[t], gsem)

    def quad(g, carry):
        for q in range(NRING):
            t = NRING * g + q
            pltpu.make_async_copy(
                feat_hbm.at[avc.at[0]], nbuf.at[q], gsem).wait()

            @pl.when(t + (NRING - 1) < NSTREAM)
            def _():
                pltpu.async_copy(
                    feat_hbm.at[avc.at[t + (NRING - 1)]],
                    nbuf.at[(q + NRING - 1) % NRING], gsem)

            lr4 = lax.rem(t, 16) * 4
            for nl in range(4):
                def red(k, accs):
                    return tuple(
                        accs[j] + nbuf[q, nl * 32 + k, pl.ds(j * 16, 16)]
                        for j in range(8)
                    )
                accs = tuple(jnp.zeros((16,), jnp.float32) for _ in range(8))
                accs = lax.fori_loop(0, 32, red, accs)
                for j in range(8):
                    sumbuf[lr4 + nl, pl.ds(j * 16, 16)] = accs[j]

            @pl.when(lax.rem(t, 16) == 15)
            def _():
                pltpu.async_copy(
                    sumbuf,
                    sum_hbm.at[pl.ds(pl.multiple_of(base + (t - 15) * 4, 64), NB)],
                    wsem).wait()
        return carry

    lax.fori_loop(0, NSTREAM // NRING, quad, 0)


@functools.lru_cache(maxsize=1)
def _build_sc_kernel():
    mesh = plsc.VectorSubcoreMesh(core_axis_name="c", subcore_axis_name="s")
    return pl.kernel(
        _sc_sage_body,
        out_type=[
            jax.ShapeDtypeStruct((BP, D), jnp.float32),
            jax.ShapeDtypeStruct((BP, D), jnp.float32),
        ],
        mesh=mesh,
        scratch_types=[
            pltpu.VMEM((BPW,), jnp.int32),
            pltpu.VMEM((NB, 128), jnp.int32),
            pltpu.VMEM((NSTREAM, 128), jnp.int32),
            pltpu.VMEM((NB, D), jnp.float32),
            pltpu.VMEM((NRING, 128, D), jnp.float32),
            pltpu.VMEM((NB, D), jnp.float32),
            pltpu.SemaphoreType.DMA,
            pltpu.SemaphoreType.DMA,
        ],
    )


_RB = 512


def _tc_body(self_ref, sum_ref, w1a_ref, w1s_ref, b1_ref, w3_ref, b3_ref, out_ref):
    x = jnp.dot(self_ref[...], w1a_ref[...], preferred_element_type=jnp.float32)
    x = x + jnp.dot(sum_ref[...], w1s_ref[...], preferred_element_type=jnp.float32)
    x = jnp.maximum(x + b1_ref[...], 0.0)
    l = jnp.dot(x, w3_ref[...], preferred_element_type=jnp.float32) + b3_ref[...]
    ss = jnp.sum(l * l, axis=1, keepdims=True)
    denom = jnp.maximum(jnp.sqrt(ss), 1e-12)
    out_ref[...] = l / denom


def kernel(inputs, adj, feat_data, W1, b1, W3, b3):
    B = inputs.shape[0]
    inputs_p = jnp.concatenate(
        [inputs.astype(jnp.int32), jnp.zeros((BP - B,), jnp.int32)])
    adj_p = jnp.pad(adj, ((0, 0), (0, 128 - DEG)))

    self_feat, sums = _build_sc_kernel()(inputs_p, adj_p, feat_data)

    w1a_t = W1[:, :D].T
    w1s_t = (W1[:, D:] * (1.0 / DEG)).T
    w3_t = jnp.pad(W3.T, ((0, 0), (0, 128 - N_CLASSES)))
    b1_r = b1.reshape(1, OUT_DIM)
    b3_r = jnp.pad(b3, (0, 128 - N_CLASSES)).reshape(1, 128)

    logits = pl.pallas_call(
        _tc_body,
        out_shape=jax.ShapeDtypeStruct((BP, 128), jnp.float32),
        grid=(BP // _RB,),
        in_specs=[
            pl.BlockSpec((_RB, D), lambda i: (i, 0)),
            pl.BlockSpec((_RB, D), lambda i: (i, 0)),
            pl.BlockSpec((D, OUT_DIM), lambda i: (0, 0)),
            pl.BlockSpec((D, OUT_DIM), lambda i: (0, 0)),
            pl.BlockSpec((1, OUT_DIM), lambda i: (0, 0)),
            pl.BlockSpec((OUT_DIM, 128), lambda i: (0, 0)),
            pl.BlockSpec((1, 128), lambda i: (0, 0)),
        ],
        out_specs=pl.BlockSpec((_RB, 128), lambda i: (i, 0)),
    )(self_feat, sums, w1a_t, w1s_t, b1_r, w3_t, b3_r)

    return logits[:B, :N_CLASSES]
